# baseline (speedup 1.0000x reference)
import functools

import jax
import jax.numpy as jnp
from jax import lax
from jax.experimental import pallas as pl
from jax.experimental.pallas import tpu as pltpu
from jax.experimental.pallas import tpu_sc as plsc

N = 10000
D = 128
E = 320000
NC = 2
NS = 16
NT = NC * NS
K = 128
NCH = 80
KH = K // 2
NB = 5
EPAD = NT * NCH * K
NROW = N + 8
STRIPE = 624
STRIPE_LAST = N - STRIPE * (NS - 1)
CW = 128


def _stripes(sid, mk):
    @pl.when(sid < NS - 1)
    def _():
        mk(sid * STRIPE, STRIPE)

    @pl.when(sid == NS - 1)
    def _():
        mk(sid * STRIPE, STRIPE_LAST)


@functools.lru_cache(maxsize=None)
def _make_seg_kernel():
    mesh = plsc.VectorSubcoreMesh(core_axis_name="c", subcore_axis_name="s",
                                  num_cores=NC, num_subcores=NS)
    out_type = [jax.ShapeDtypeStruct((NC, N, D), jnp.float32)]
    scratch = [
        pltpu.VMEM_SHARED((NROW, D), jnp.float32),
        pltpu.VMEM((NCH, K), jnp.int32),
        pltpu.VMEM((3, K), jnp.int32),
        pltpu.VMEM((2, K, D), jnp.float32),
        pltpu.SemaphoreType.DMA,
        pltpu.SemaphoreType.DMA,
        pltpu.SemaphoreType.DMA,
    ]

    def gather_halves(z_hbm, src_v, rows_v, c, sb, sem_g):
        for h in range(2):
            pltpu.async_copy(
                z_hbm.at[src_v.at[c, pl.ds(h * KH, KH)]],
                rows_v.at[sb, pl.ds(h * KH, KH)], sem_g)

    def body(z_hbm, src_hbm, dst_hbm, zer_hbm, out_s,
             s_sh, src_v, dst_v, rows_v, sem_g, sem_i, sem_s):
        cid = lax.axis_index("c")
        sid = lax.axis_index("s")
        wid = cid * NS + sid

        def load_idx(c, slot):
            pltpu.async_copy(dst_hbm.at[wid, c], dst_v.at[slot], sem_i)

        def wait_idx(c, slot):
            pltpu.make_async_copy(dst_hbm.at[wid, c], dst_v.at[slot],
                                  sem_i).wait()

        pltpu.sync_copy(src_hbm.at[wid], src_v)
        load_idx(0, 0)
        load_idx(1, 1)

        _stripes(sid, lambda r0, sz: pltpu.sync_copy(
            zer_hbm.at[pl.ds(r0, sz)], s_sh.at[pl.ds(r0, sz)]))

        wait_idx(0, 0)
        gather_halves(z_hbm, src_v, rows_v, 0, 0, sem_g)
        plsc.subcore_barrier()

        @pl.loop(0, NCH, step=2)
        def _chunks(jj):
            for b in range(2):
                c = jj + b
                si = c % 3
                for h in range(2):
                    pltpu.make_async_copy(
                        z_hbm.at[src_v.at[c, pl.ds(h * KH, KH)]],
                        rows_v.at[b, pl.ds(h * KH, KH)], sem_g).wait()
                pltpu.async_copy(rows_v.at[b], s_sh.at[dst_v.at[si]],
                                 sem_s, add=True)

                @pl.when(c > 0)
                def _():
                    pltpu.make_async_copy(
                        rows_v.at[1 - b], s_sh.at[dst_v.at[(c - 1) % 3]],
                        sem_s).wait()

                @pl.when(c + 2 < NCH)
                def _():
                    load_idx(c + 2, (c + 2) % 3)

                @pl.when(c + 1 < NCH)
                def _():
                    wait_idx(c + 1, (c + 1) % 3)
                    gather_halves(z_hbm, src_v, rows_v, c + 1, 1 - b,
                                  sem_g)

        pltpu.make_async_copy(rows_v.at[(NCH - 1) % 2],
                              s_sh.at[dst_v.at[(NCH - 1) % 3]],
                              sem_s).wait()

        plsc.subcore_barrier()
        _stripes(sid, lambda r0, sz: pltpu.sync_copy(
            s_sh.at[pl.ds(r0, sz)], out_s.at[cid, pl.ds(r0, sz)]))

    return pl.kernel(body, out_type=out_type, mesh=mesh,
                     scratch_types=scratch)


@functools.lru_cache(maxsize=None)
def _make_cnt_kernel():
    mesh = plsc.VectorSubcoreMesh(core_axis_name="c", subcore_axis_name="s",
                                  num_cores=NC, num_subcores=NS)
    out_type = [jax.ShapeDtypeStruct((NC, N, CW), jnp.float32)]
    scratch = [
        pltpu.VMEM_SHARED((NROW, CW), jnp.float32),
        pltpu.VMEM((2, K), jnp.int32),
        pltpu.VMEM((K, CW), jnp.float32),
        pltpu.SemaphoreType.DMA,
    ]

    def body(dst_hbm, zer_hbm, ones_hbm, out_c, cnt_sh, dst_v, ones_v, sem):
        cid = lax.axis_index("c")
        sid = lax.axis_index("s")
        wid = cid * NS + sid

        pltpu.sync_copy(ones_hbm, ones_v)
        pltpu.sync_copy(dst_hbm.at[wid, 0], dst_v.at[0])
        pltpu.async_copy(dst_hbm.at[wid, 1], dst_v.at[1], sem)
        _stripes(sid, lambda r0, sz: pltpu.sync_copy(
            zer_hbm.at[pl.ds(r0, sz)], cnt_sh.at[pl.ds(r0, sz)]))
        plsc.subcore_barrier()

        @pl.loop(0, NCH, step=2)
        def _chunks(jj):
            for b in range(2):
                j = jj + b

                @pl.when(j > 0)
                def _():
                    pltpu.make_async_copy(
                        dst_hbm.at[wid, j], dst_v.at[b], sem).wait()
                pltpu.sync_copy(ones_v, cnt_sh.at[dst_v.at[b]], add=True)

                @pl.when(j < NCH - 2)
                def _():
                    pltpu.async_copy(dst_hbm.at[wid, j + 2], dst_v.at[b], sem)

        plsc.subcore_barrier()
        _stripes(sid, lambda r0, sz: pltpu.sync_copy(
            cnt_sh.at[pl.ds(r0, sz)], out_c.at[cid, pl.ds(r0, sz)]))

    return pl.kernel(body, out_type=out_type, mesh=mesh,
                     scratch_types=scratch)


R = 2000


def _mm_body(x_ref, w_ref, o_ref):
    o_ref[...] = jnp.dot(x_ref[...], w_ref[...],
                         preferred_element_type=jnp.float32)


_mm = pl.pallas_call(
    _mm_body,
    grid=(N // R,),
    in_specs=[pl.BlockSpec((R, D), lambda i: (i, 0)),
              pl.BlockSpec((D, D), lambda i: (0, 0))],
    out_specs=pl.BlockSpec((R, D), lambda i: (i, 0)),
    out_shape=jax.ShapeDtypeStruct((N, D), jnp.float32),
)


def _fused_body(with_next, sp_ref, cp_ref, h_ref, wr_ref, bl_ref, g_ref,
                b_ref, *rest):
    if with_next:
        wl_ref, h_out, z_out = rest
    else:
        (h_out,) = rest
    s = sp_ref[0] + sp_ref[1]
    cnt = cp_ref[0, :, 0:1] + cp_ref[1, :, 0:1]
    inv = 1.0 / jnp.maximum(cnt, 1.0)
    t = s * inv + bl_ref[...] + jnp.dot(h_ref[...], wr_ref[...],
                                        preferred_element_type=jnp.float32)
    mu = jnp.mean(t, axis=-1, keepdims=True)
    c = t - mu
    var = jnp.mean(c * c, axis=-1, keepdims=True)
    hp = jnp.maximum(c * lax.rsqrt(var + 1e-5) * g_ref[...] + b_ref[...], 0.0)
    h_out[...] = hp
    if with_next:
        z_out[...] = jnp.dot(hp, wl_ref[...],
                             preferred_element_type=jnp.float32)


def _make_fused(with_next: bool):
    in_specs = [
        pl.BlockSpec((NC, R, D), lambda i: (0, i, 0)),
        pl.BlockSpec((NC, R, CW), lambda i: (0, i, 0)),
        pl.BlockSpec((R, D), lambda i: (i, 0)),
        pl.BlockSpec((D, D), lambda i: (0, 0)),
        pl.BlockSpec((1, D), lambda i: (0, 0)),
        pl.BlockSpec((1, D), lambda i: (0, 0)),
        pl.BlockSpec((1, D), lambda i: (0, 0)),
    ]
    out_shape = [jax.ShapeDtypeStruct((N, D), jnp.float32)]
    out_specs = [pl.BlockSpec((R, D), lambda i: (i, 0))]
    if with_next:
        in_specs.append(pl.BlockSpec((D, D), lambda i: (0, 0)))
        out_shape.append(jax.ShapeDtypeStruct((N, D), jnp.float32))
        out_specs.append(pl.BlockSpec((R, D), lambda i: (i, 0)))
    return pl.pallas_call(
        functools.partial(_fused_body, with_next),
        grid=(N // R,),
        in_specs=in_specs,
        out_specs=out_specs,
        out_shape=out_shape,
    )


_fused_next = _make_fused(True)
_fused_last = _make_fused(False)


def _seg(z, src_r, dst_r, zeros_nd):
    return _make_seg_kernel()(z, src_r, dst_r, zeros_nd)


def _cnt(dst_r, zeros_c, ones_c):
    return _make_cnt_kernel()(dst_r, zeros_c, ones_c)


def kernel(x, edge_index, Wl0, bl0, Wr0, g0, b0, Wl1, bl1, Wr1, g1, b1,
           Wl2, bl2, Wr2, g2, b2):
    ppt = (EPAD - E) // NT
    pad_dst = jnp.broadcast_to(N + (jnp.arange(ppt, dtype=jnp.int32) % 8),
                               (NT, ppt))
    src_r = jnp.concatenate(
        [edge_index[0].reshape(NT, E // NT),
         jnp.zeros((NT, ppt), jnp.int32)], axis=1).reshape(NT, NCH, K)
    dst_r = jnp.concatenate(
        [edge_index[1].reshape(NT, E // NT), pad_dst],
        axis=1).reshape(NT, NCH, K)
    zeros_nd = jnp.zeros((N, D), jnp.float32)
    zeros_c = jnp.zeros((N, CW), jnp.float32)
    ones_c = jnp.ones((K, CW), jnp.float32)
    bl0r, g0r, b0r = bl0.reshape(1, D), g0.reshape(1, D), b0.reshape(1, D)
    bl1r, g1r, b1r = bl1.reshape(1, D), g1.reshape(1, D), b1.reshape(1, D)
    bl2r, g2r, b2r = bl2.reshape(1, D), g2.reshape(1, D), b2.reshape(1, D)

    (cp,) = _cnt(dst_r, zeros_c, ones_c)
    z0 = _mm(x, Wl0)
    (sp0,) = _seg(z0, src_r, dst_r, zeros_nd)
    h1, z1 = _fused_next(sp0, cp, x, Wr0, bl0r, g0r, b0r, Wl1)
    (sp1,) = _seg(z1, src_r, dst_r, zeros_nd)
    h2, z2 = _fused_next(sp1, cp, h1, Wr1, bl1r, g1r, b1r, Wl2)
    (sp2,) = _seg(z2, src_r, dst_r, zeros_nd)
    (h3,) = _fused_last(sp2, cp, h2, Wr2, bl2r, g2r, b2r)
    return h3


# speedup vs baseline: 1.0598x; 1.0598x over previous
import functools

import jax
import jax.numpy as jnp
from jax import lax
from jax.experimental import pallas as pl
from jax.experimental.pallas import tpu as pltpu
from jax.experimental.pallas import tpu_sc as plsc

N = 10000
D = 128
E = 320000
NC = 2
NS = 16
NT = NC * NS
K = 64
NCH = 160
NB = 5
EPAD = NT * NCH * K
NROW = N + 8
STRIPE = 624
STRIPE_LAST = N - STRIPE * (NS - 1)
CW = 128


def _stripes(sid, mk):
    @pl.when(sid < NS - 1)
    def _():
        mk(sid * STRIPE, STRIPE)

    @pl.when(sid == NS - 1)
    def _():
        mk(sid * STRIPE, STRIPE_LAST)


@functools.lru_cache(maxsize=None)
def _make_seg_kernel():
    mesh = plsc.VectorSubcoreMesh(core_axis_name="c", subcore_axis_name="s",
                                  num_cores=NC, num_subcores=NS)
    out_type = [jax.ShapeDtypeStruct((NC, N, D), jnp.float32)]
    scratch = [
        pltpu.VMEM_SHARED((NROW, D), jnp.float32),
        pltpu.VMEM((NB, K), jnp.int32),
        pltpu.VMEM((NB, K), jnp.int32),
        pltpu.VMEM((NB, K, D), jnp.float32),
        pltpu.SemaphoreType.DMA,
        pltpu.SemaphoreType.DMA,
    ]

    def body(z_hbm, src_hbm, dst_hbm, zer_hbm, out_s,
             s_sh, src_v, dst_v, rows_v, sem_g, sem_i):
        cid = lax.axis_index("c")
        sid = lax.axis_index("s")
        wid = cid * NS + sid

        for b in range(NB):
            pltpu.async_copy(src_hbm.at[wid, b], src_v.at[b], sem_i)
            pltpu.async_copy(dst_hbm.at[wid, b], dst_v.at[b], sem_i)

        _stripes(sid, lambda r0, sz: pltpu.sync_copy(
            zer_hbm.at[pl.ds(r0, sz)], s_sh.at[pl.ds(r0, sz)]))

        for b in range(NB - 1):
            pltpu.make_async_copy(src_hbm.at[wid, b], src_v.at[b],
                                  sem_i).wait()
            pltpu.make_async_copy(dst_hbm.at[wid, b], dst_v.at[b],
                                  sem_i).wait()
            pltpu.async_copy(z_hbm.at[src_v.at[b]], rows_v.at[b], sem_g)
        plsc.subcore_barrier()

        @pl.loop(0, NCH, step=NB)
        def _chunks(jj):
            for b in range(NB):
                c = jj + b
                pltpu.make_async_copy(
                    z_hbm.at[src_v.at[b]], rows_v.at[b], sem_g).wait()
                pltpu.sync_copy(rows_v.at[b], s_sh.at[dst_v.at[b]], add=True)

                @pl.when(c + NB < NCH)
                def _():
                    pltpu.async_copy(src_hbm.at[wid, c + NB], src_v.at[b],
                                     sem_i)
                    pltpu.async_copy(dst_hbm.at[wid, c + NB], dst_v.at[b],
                                     sem_i)

                bw = (b + NB - 1) % NB
                @pl.when(c + NB - 1 < NCH)
                def _():
                    pltpu.make_async_copy(
                        src_hbm.at[wid, c + NB - 1], src_v.at[bw],
                        sem_i).wait()
                    pltpu.make_async_copy(
                        dst_hbm.at[wid, c + NB - 1], dst_v.at[bw],
                        sem_i).wait()
                    pltpu.async_copy(z_hbm.at[src_v.at[bw]], rows_v.at[bw],
                                     sem_g)

        plsc.subcore_barrier()
        _stripes(sid, lambda r0, sz: pltpu.sync_copy(
            s_sh.at[pl.ds(r0, sz)], out_s.at[cid, pl.ds(r0, sz)]))

    return pl.kernel(body, out_type=out_type, mesh=mesh,
                     scratch_types=scratch)


@functools.lru_cache(maxsize=None)
def _make_cnt_kernel():
    mesh = plsc.VectorSubcoreMesh(core_axis_name="c", subcore_axis_name="s",
                                  num_cores=NC, num_subcores=NS)
    out_type = [jax.ShapeDtypeStruct((NC, N, CW), jnp.float32)]
    scratch = [
        pltpu.VMEM_SHARED((NROW, CW), jnp.float32),
        pltpu.VMEM((2, K), jnp.int32),
        pltpu.VMEM((K, CW), jnp.float32),
        pltpu.SemaphoreType.DMA,
    ]

    def body(dst_hbm, zer_hbm, ones_hbm, out_c, cnt_sh, dst_v, ones_v, sem):
        cid = lax.axis_index("c")
        sid = lax.axis_index("s")
        wid = cid * NS + sid

        pltpu.sync_copy(ones_hbm, ones_v)
        pltpu.sync_copy(dst_hbm.at[wid, 0], dst_v.at[0])
        pltpu.async_copy(dst_hbm.at[wid, 1], dst_v.at[1], sem)
        _stripes(sid, lambda r0, sz: pltpu.sync_copy(
            zer_hbm.at[pl.ds(r0, sz)], cnt_sh.at[pl.ds(r0, sz)]))
        plsc.subcore_barrier()

        @pl.loop(0, NCH, step=2)
        def _chunks(jj):
            for b in range(2):
                j = jj + b

                @pl.when(j > 0)
                def _():
                    pltpu.make_async_copy(
                        dst_hbm.at[wid, j], dst_v.at[b], sem).wait()
                pltpu.sync_copy(ones_v, cnt_sh.at[dst_v.at[b]], add=True)

                @pl.when(j < NCH - 2)
                def _():
                    pltpu.async_copy(dst_hbm.at[wid, j + 2], dst_v.at[b], sem)

        plsc.subcore_barrier()
        _stripes(sid, lambda r0, sz: pltpu.sync_copy(
            cnt_sh.at[pl.ds(r0, sz)], out_c.at[cid, pl.ds(r0, sz)]))

    return pl.kernel(body, out_type=out_type, mesh=mesh,
                     scratch_types=scratch)


R = 2000


def _mm_body(x_ref, w_ref, o_ref):
    o_ref[...] = jnp.dot(x_ref[...], w_ref[...],
                         preferred_element_type=jnp.float32)


_mm = pl.pallas_call(
    _mm_body,
    grid=(N // R,),
    in_specs=[pl.BlockSpec((R, D), lambda i: (i, 0)),
              pl.BlockSpec((D, D), lambda i: (0, 0))],
    out_specs=pl.BlockSpec((R, D), lambda i: (i, 0)),
    out_shape=jax.ShapeDtypeStruct((N, D), jnp.float32),
)


def _fused_body(with_next, sp_ref, cp_ref, h_ref, wr_ref, bl_ref, g_ref,
                b_ref, *rest):
    if with_next:
        wl_ref, h_out, z_out = rest
    else:
        (h_out,) = rest
    s = sp_ref[0] + sp_ref[1]
    cnt = cp_ref[0, :, 0:1] + cp_ref[1, :, 0:1]
    inv = 1.0 / jnp.maximum(cnt, 1.0)
    t = s * inv + bl_ref[...] + jnp.dot(h_ref[...], wr_ref[...],
                                        preferred_element_type=jnp.float32)
    mu = jnp.mean(t, axis=-1, keepdims=True)
    c = t - mu
    var = jnp.mean(c * c, axis=-1, keepdims=True)
    hp = jnp.maximum(c * lax.rsqrt(var + 1e-5) * g_ref[...] + b_ref[...], 0.0)
    h_out[...] = hp
    if with_next:
        z_out[...] = jnp.dot(hp, wl_ref[...],
                             preferred_element_type=jnp.float32)


def _make_fused(with_next: bool):
    in_specs = [
        pl.BlockSpec((NC, R, D), lambda i: (0, i, 0)),
        pl.BlockSpec((NC, R, CW), lambda i: (0, i, 0)),
        pl.BlockSpec((R, D), lambda i: (i, 0)),
        pl.BlockSpec((D, D), lambda i: (0, 0)),
        pl.BlockSpec((1, D), lambda i: (0, 0)),
        pl.BlockSpec((1, D), lambda i: (0, 0)),
        pl.BlockSpec((1, D), lambda i: (0, 0)),
    ]
    out_shape = [jax.ShapeDtypeStruct((N, D), jnp.float32)]
    out_specs = [pl.BlockSpec((R, D), lambda i: (i, 0))]
    if with_next:
        in_specs.append(pl.BlockSpec((D, D), lambda i: (0, 0)))
        out_shape.append(jax.ShapeDtypeStruct((N, D), jnp.float32))
        out_specs.append(pl.BlockSpec((R, D), lambda i: (i, 0)))
    return pl.pallas_call(
        functools.partial(_fused_body, with_next),
        grid=(N // R,),
        in_specs=in_specs,
        out_specs=out_specs,
        out_shape=out_shape,
    )


_fused_next = _make_fused(True)
_fused_last = _make_fused(False)


def _seg(z, src_r, dst_r, zeros_nd):
    return _make_seg_kernel()(z, src_r, dst_r, zeros_nd)


def _cnt(dst_r, zeros_c, ones_c):
    return _make_cnt_kernel()(dst_r, zeros_c, ones_c)


def kernel(x, edge_index, Wl0, bl0, Wr0, g0, b0, Wl1, bl1, Wr1, g1, b1,
           Wl2, bl2, Wr2, g2, b2):
    ppt = (EPAD - E) // NT
    pad_dst = jnp.broadcast_to(N + (jnp.arange(ppt, dtype=jnp.int32) % 8),
                               (NT, ppt))
    src_r = jnp.concatenate(
        [edge_index[0].reshape(NT, E // NT),
         jnp.zeros((NT, ppt), jnp.int32)], axis=1).reshape(NT, NCH, K)
    dst_r = jnp.concatenate(
        [edge_index[1].reshape(NT, E // NT), pad_dst],
        axis=1).reshape(NT, NCH, K)
    zeros_nd = jnp.zeros((N, D), jnp.float32)
    zeros_c = jnp.zeros((N, CW), jnp.float32)
    ones_c = jnp.ones((K, CW), jnp.float32)
    bl0r, g0r, b0r = bl0.reshape(1, D), g0.reshape(1, D), b0.reshape(1, D)
    bl1r, g1r, b1r = bl1.reshape(1, D), g1.reshape(1, D), b1.reshape(1, D)
    bl2r, g2r, b2r = bl2.reshape(1, D), g2.reshape(1, D), b2.reshape(1, D)

    (cp,) = _cnt(dst_r, zeros_c, ones_c)
    z0 = _mm(x, Wl0)
    (sp0,) = _seg(z0, src_r, dst_r, zeros_nd)
    h1, z1 = _fused_next(sp0, cp, x, Wr0, bl0r, g0r, b0r, Wl1)
    (sp1,) = _seg(z1, src_r, dst_r, zeros_nd)
    h2, z2 = _fused_next(sp1, cp, h1, Wr1, bl1r, g1r, b1r, Wl2)
    (sp2,) = _seg(z2, src_r, dst_r, zeros_nd)
    (h3,) = _fused_last(sp2, cp, h2, Wr2, bl2r, g2r, b2r)
    return h3
